# baseline (speedup 1.0000x reference)
import functools

import jax
import jax.numpy as jnp
from jax import lax
from jax.experimental import pallas as pl
from jax.experimental.pallas import tpu as pltpu
from jax.experimental.pallas import tpu_sc as plsc

N_RAYS = 65536
N_BINS = 128
N_SAMPLES = 128
L = 16
NC, NS = 2, 16
NW = NC * NS
RPW = N_RAYS // NW
R = 64
NCHUNK = RPW // R
NV = N_BINS // L


def _vperm(v, i):
    return lax.gather(
        v, i[:, None],
        dimension_numbers=lax.GatherDimensionNumbers(
            offset_dims=(), collapsed_slice_dims=(0,), start_index_map=(0,)),
        slice_sizes=(1,),
        mode=lax.GatherScatterMode.PROMISE_IN_BOUNDS)


def _body(bins_hbm, w_hbm, out_hbm, w_bufs, b_bufs, c_buf, o_bufs,
          in_sems, out_sems):
    wid = lax.axis_index("s") * NC + lax.axis_index("c")
    base = wid * RPW

    inv256 = jnp.float32(1.0 / 256.0)
    lane = lax.iota(jnp.int32, L)

    def start_in(ci, slot):
        row0 = base + ci * R
        pltpu.async_copy(w_hbm.at[pl.ds(row0, R)], w_bufs.at[slot],
                         in_sems.at[slot])
        pltpu.async_copy(bins_hbm.at[pl.ds(row0, R)], b_bufs.at[slot],
                         in_sems.at[slot])

    def wait_in(slot):
        pltpu.make_async_copy(w_hbm.at[pl.ds(base, R)], w_bufs.at[slot],
                              in_sems.at[slot]).wait()
        pltpu.make_async_copy(bins_hbm.at[pl.ds(base, R)], b_bufs.at[slot],
                              in_sems.at[slot]).wait()

    def wait_out(slot):
        pltpu.make_async_copy(o_bufs.at[slot], out_hbm.at[pl.ds(base, R)],
                              out_sems.at[slot]).wait()

    start_in(0, 0)

    @pl.loop(0, NCHUNK, step=2)
    def _chunk_pair(ci):
        for slot in range(2):
            cur = ci + slot
            w_buf, b_buf, o_buf = (w_bufs.at[slot], b_bufs.at[slot],
                                   o_bufs.at[slot])

            @pl.when(cur + 1 < NCHUNK)
            def _():
                start_in(cur + 1, 1 - slot)

            wait_in(slot)

            @pl.when(cur >= 2)
            def _():
                wait_out(slot)

            _do_chunk(lane, inv256, w_buf, b_buf, c_buf, o_buf)
            pltpu.async_copy(o_buf, out_hbm.at[pl.ds(base + cur * R, R)],
                             out_sems.at[slot])

    wait_out(0)
    wait_out(1)


def _do_chunk(lane, inv256, w_buf, b_buf, c_buf, o_buf):
        NT = N_SAMPLES // L

        lane15 = jnp.full((L,), L - 1, jnp.int32)

        @plsc.parallel_loop(0, R, unroll=4)
        def _ray_cdf(r):
            wv = [w_buf[r, pl.ds(L * k, L)] + 1e-5 for k in range(NV)]
            cums = [plsc.cumsum(wv[k]) for k in range(NV)]
            acc = [cums[0]]
            for k in range(1, NV):
                acc.append(cums[k] + _vperm(acc[k - 1], lane15))
            inv_total = jnp.ones((L,), jnp.float32) / _vperm(acc[NV - 1], lane15)
            for k in range(NV):
                c_buf[r, pl.ds(L * k, L)] = acc[k] * inv_total

        us = [(lane * 2 + (32 * t + 1)).astype(jnp.float32) * inv256
              for t in range(NT)]

        @plsc.parallel_loop(0, R, unroll=3)
        def _ray(r):
            rowv = jnp.full((L,), r, jnp.int32)
            ends = plsc.load_gather(c_buf, [rowv, lane * 8 + 7])
            blk = [jnp.zeros((L,), jnp.int32) for _ in range(NT)]
            for s in (8, 4, 2, 1):
                es = [_vperm(ends, blk[t] + (s - 1)) for t in range(NT)]
                blk = [blk[t] + jnp.where(es[t] <= us[t], s, 0)
                       for t in range(NT)]
            idx = [blk[t] * 8 for t in range(NT)]
            for s in (4, 2):
                gs = [plsc.load_gather(c_buf, [rowv, idx[t] + (s - 1)])
                      for t in range(NT)]
                idx = [idx[t] + jnp.where(gs[t] <= us[t], s, 0)
                       for t in range(NT)]
            g1s = [plsc.load_gather(c_buf, [rowv, idx[t]]) for t in range(NT)]
            adv = [g1s[t] <= us[t] for t in range(NT)]
            idx = [idx[t] + jnp.where(adv[t], 1, 0) for t in range(NT)]
            oth = [jnp.where(adv[t], idx[t],
                             jnp.maximum(idx[t] - 1, 0)) for t in range(NT)]
            gos = [plsc.load_gather(c_buf, [rowv, oth[t]]) for t in range(NT)]
            b0s = [plsc.load_gather(b_buf, [rowv, idx[t]]) for t in range(NT)]
            b1s = [plsc.load_gather(b_buf, [rowv, idx[t] + 1])
                   for t in range(NT)]
            for t in range(NT):
                cdf0 = jnp.where(adv[t], g1s[t], gos[t])
                cdf0 = jnp.where(idx[t] > 0, cdf0, 0.0)
                cdf1 = jnp.where(adv[t], gos[t], g1s[t])
                denom = cdf1 - cdf0
                denom = jnp.where(denom < 1e-5, 1.0, denom)
                tt = (us[t] - cdf0) / denom
                o_buf[r, pl.ds(L * t, L)] = b0s[t] + tt * (b1s[t] - b0s[t])


@jax.jit
def _sample_pdf_sc(bins, weights):
    mesh = plsc.VectorSubcoreMesh(core_axis_name="c", subcore_axis_name="s")
    kfn = pl.kernel(
        _body,
        out_type=jax.ShapeDtypeStruct((N_RAYS, N_SAMPLES), jnp.float32),
        mesh=mesh,
        scratch_types=[
            pltpu.VMEM((2, R, N_BINS), jnp.float32),
            pltpu.VMEM((2, R, N_BINS + 1), jnp.float32),
            pltpu.VMEM((R, N_BINS), jnp.float32),
            pltpu.VMEM((2, R, N_SAMPLES), jnp.float32),
            pltpu.SemaphoreType.DMA((2,)),
            pltpu.SemaphoreType.DMA((2,)),
        ],
        compiler_params=pltpu.CompilerParams(needs_layout_passes=False),
    )
    return kfn(bins, weights)


def kernel(bins, weights, n_samples):
    del n_samples
    return _sample_pdf_sc(bins, weights)


# speedup vs baseline: 1.0045x; 1.0045x over previous
import functools

import jax
import jax.numpy as jnp
from jax import lax
from jax.experimental import pallas as pl
from jax.experimental.pallas import tpu as pltpu
from jax.experimental.pallas import tpu_sc as plsc

N_RAYS = 65536
N_BINS = 128
N_SAMPLES = 128
L = 16
NC, NS = 2, 16
NW = NC * NS
RPW = N_RAYS // NW
R = 64
NCHUNK = RPW // R
NV = N_BINS // L


def _vperm(v, i):
    return lax.gather(
        v, i[:, None],
        dimension_numbers=lax.GatherDimensionNumbers(
            offset_dims=(), collapsed_slice_dims=(0,), start_index_map=(0,)),
        slice_sizes=(1,),
        mode=lax.GatherScatterMode.PROMISE_IN_BOUNDS)


def _body(bins_hbm, w_hbm, out_hbm, w_bufs, b_bufs, c_buf, o_bufs,
          in_sems, out_sems):
    wid = lax.axis_index("s") * NC + lax.axis_index("c")
    base = wid * RPW

    inv256 = jnp.float32(1.0 / 256.0)
    lane = lax.iota(jnp.int32, L)

    def start_in(ci, slot):
        row0 = base + ci * R
        pltpu.async_copy(w_hbm.at[pl.ds(row0, R)], w_bufs.at[slot],
                         in_sems.at[slot])
        pltpu.async_copy(bins_hbm.at[pl.ds(row0, R)], b_bufs.at[slot],
                         in_sems.at[slot])

    def wait_in(slot):
        pltpu.make_async_copy(w_hbm.at[pl.ds(base, R)], w_bufs.at[slot],
                              in_sems.at[slot]).wait()
        pltpu.make_async_copy(bins_hbm.at[pl.ds(base, R)], b_bufs.at[slot],
                              in_sems.at[slot]).wait()

    def wait_out(slot):
        pltpu.make_async_copy(o_bufs.at[slot], out_hbm.at[pl.ds(base, R)],
                              out_sems.at[slot]).wait()

    start_in(0, 0)

    @pl.loop(0, NCHUNK, step=2)
    def _chunk_pair(ci):
        for slot in range(2):
            cur = ci + slot
            w_buf, b_buf, o_buf = (w_bufs.at[slot], b_bufs.at[slot],
                                   o_bufs.at[slot])

            @pl.when(cur + 1 < NCHUNK)
            def _():
                start_in(cur + 1, 1 - slot)

            wait_in(slot)

            @pl.when(cur >= 2)
            def _():
                wait_out(slot)

            _do_chunk(lane, inv256, w_buf, b_buf, c_buf, o_buf)
            pltpu.async_copy(o_buf, out_hbm.at[pl.ds(base + cur * R, R)],
                             out_sems.at[slot])

    wait_out(0)
    wait_out(1)


def _do_chunk(lane, inv256, w_buf, b_buf, c_buf, o_buf):
        NT = N_SAMPLES // L

        lane15 = jnp.full((L,), L - 1, jnp.int32)

        @plsc.parallel_loop(0, R, unroll=4)
        def _ray_cdf(r):
            wv = [w_buf[r, pl.ds(L * k, L)] + 1e-5 for k in range(NV)]
            cums = [plsc.cumsum(wv[k]) for k in range(NV)]
            acc = [cums[0]]
            for k in range(1, NV):
                acc.append(cums[k] + _vperm(acc[k - 1], lane15))
            inv_total = jnp.ones((L,), jnp.float32) / _vperm(acc[NV - 1], lane15)
            for k in range(NV):
                c_buf[r, pl.ds(L * k, L)] = acc[k] * inv_total

        us = [(lane * 2 + (32 * t + 1)).astype(jnp.float32) * inv256
              for t in range(NT)]

        @plsc.parallel_loop(0, R, unroll=2)
        def _ray(r):
            rowv = jnp.full((L,), r, jnp.int32)
            ends = plsc.load_gather(c_buf, [rowv, lane * 8 + 7])
            blk = [jnp.zeros((L,), jnp.int32) for _ in range(NT)]
            for s in (8, 4, 2, 1):
                es = [_vperm(ends, blk[t] + (s - 1)) for t in range(NT)]
                blk = [blk[t] + jnp.where(es[t] <= us[t], s, 0)
                       for t in range(NT)]
            idx = [blk[t] * 8 for t in range(NT)]
            for s in (4, 2):
                gs = [plsc.load_gather(c_buf, [rowv, idx[t] + (s - 1)])
                      for t in range(NT)]
                idx = [idx[t] + jnp.where(gs[t] <= us[t], s, 0)
                       for t in range(NT)]
            g1s = [plsc.load_gather(c_buf, [rowv, idx[t]]) for t in range(NT)]
            adv = [g1s[t] <= us[t] for t in range(NT)]
            idx = [idx[t] + jnp.where(adv[t], 1, 0) for t in range(NT)]
            oth = [jnp.where(adv[t], idx[t],
                             jnp.maximum(idx[t] - 1, 0)) for t in range(NT)]
            gos = [plsc.load_gather(c_buf, [rowv, oth[t]]) for t in range(NT)]
            b0s = [plsc.load_gather(b_buf, [rowv, idx[t]]) for t in range(NT)]
            b1s = [plsc.load_gather(b_buf, [rowv, idx[t] + 1])
                   for t in range(NT)]
            for t in range(NT):
                cdf0 = jnp.where(adv[t], g1s[t], gos[t])
                cdf0 = jnp.where(idx[t] > 0, cdf0, 0.0)
                cdf1 = jnp.where(adv[t], gos[t], g1s[t])
                denom = cdf1 - cdf0
                denom = jnp.where(denom < 1e-5, 1.0, denom)
                tt = (us[t] - cdf0) / denom
                o_buf[r, pl.ds(L * t, L)] = b0s[t] + tt * (b1s[t] - b0s[t])


@jax.jit
def _sample_pdf_sc(bins, weights):
    mesh = plsc.VectorSubcoreMesh(core_axis_name="c", subcore_axis_name="s")
    kfn = pl.kernel(
        _body,
        out_type=jax.ShapeDtypeStruct((N_RAYS, N_SAMPLES), jnp.float32),
        mesh=mesh,
        scratch_types=[
            pltpu.VMEM((2, R, N_BINS), jnp.float32),
            pltpu.VMEM((2, R, N_BINS + 1), jnp.float32),
            pltpu.VMEM((R, N_BINS), jnp.float32),
            pltpu.VMEM((2, R, N_SAMPLES), jnp.float32),
            pltpu.SemaphoreType.DMA((2,)),
            pltpu.SemaphoreType.DMA((2,)),
        ],
        compiler_params=pltpu.CompilerParams(needs_layout_passes=False),
    )
    return kfn(bins, weights)


def kernel(bins, weights, n_samples):
    del n_samples
    return _sample_pdf_sc(bins, weights)
